# Initial kernel scaffold; baseline (speedup 1.0000x reference)
import functools

import jax
import jax.numpy as jnp
from jax import lax
from jax.experimental import pallas as pl
from jax.experimental.pallas import tpu as pltpu
from jax.experimental.pallas import tpu_sc as plsc

_NC = 2
_NS = 16
_NW = _NC * _NS

_CHUNK = 1024


@functools.partial(jax.jit, static_argnums=(2, 3))
def _gather_flat(idx_flat, table, b_per_w, n_chunks):
    d = table.shape[1]
    mesh = plsc.VectorSubcoreMesh(core_axis_name="c", subcore_axis_name="s")

    @functools.partial(
        pl.kernel,
        mesh=mesh,
        out_type=jax.ShapeDtypeStruct((idx_flat.shape[0], d), jnp.float32),
        scratch_types=[
            pltpu.VMEM((_CHUNK,), jnp.int32),
            pltpu.VMEM((_CHUNK, d), jnp.float32),
            pltpu.SemaphoreType.DMA,
        ],
    )
    def k(idx_hbm, table_hbm, out_hbm, idx_v, rows_v, sem):
        wid = lax.axis_index("s") * _NC + lax.axis_index("c")
        base = wid * b_per_w

        def body(i, _):
            off = base + i * _CHUNK
            pltpu.sync_copy(idx_hbm.at[pl.ds(off, _CHUNK)], idx_v)
            pltpu.async_copy(table_hbm.at[idx_v], rows_v, sem).wait()
            pltpu.sync_copy(rows_v, out_hbm.at[pl.ds(off, _CHUNK)])
            return 0

        lax.fori_loop(0, n_chunks, body, 0)

    return k(idx_flat, table)


def kernel(indices, table):
    n, s = indices.shape
    b_total = n * s
    assert b_total % (_NW * _CHUNK) == 0
    b_per_w = b_total // _NW
    n_chunks = b_per_w // _CHUNK
    idx_flat = indices.reshape(b_total).astype(jnp.int32)
    out = _gather_flat(idx_flat, table, b_per_w, n_chunks)
    return out.reshape(n, s, table.shape[1])


# speedup vs baseline: 4.9811x; 4.9811x over previous
import functools

import jax
import jax.numpy as jnp
from jax import lax
from jax.experimental import pallas as pl
from jax.experimental.pallas import tpu as pltpu
from jax.experimental.pallas import tpu_sc as plsc

_NC = 2
_NS = 16
_NW = _NC * _NS

_CHUNK = 1024


@functools.partial(jax.jit, static_argnums=(2, 3))
def _gather_flat(idx_flat, table, b_per_w, n_chunks):
    d = table.shape[1]
    mesh = plsc.VectorSubcoreMesh(core_axis_name="c", subcore_axis_name="s")

    @functools.partial(
        pl.kernel,
        mesh=mesh,
        out_type=jax.ShapeDtypeStruct((idx_flat.shape[0], d), jnp.float32),
        scratch_types=[
            pltpu.VMEM((_CHUNK,), jnp.int32),
            pltpu.VMEM((_CHUNK, d), jnp.float32),
            pltpu.SemaphoreType.DMA,
        ],
        compiler_params=pltpu.CompilerParams(use_tc_tiling_on_sc=False),
    )
    def k(idx_hbm, table_hbm, out_hbm, idx_v, rows_v, sem):
        wid = lax.axis_index("s") * _NC + lax.axis_index("c")
        base = wid * b_per_w

        def body(i, _):
            off = base + i * _CHUNK
            pltpu.sync_copy(idx_hbm.at[pl.ds(off, _CHUNK)], idx_v)
            pltpu.async_copy(table_hbm.at[idx_v], rows_v, sem).wait()
            pltpu.sync_copy(rows_v, out_hbm.at[pl.ds(off, _CHUNK)])
            return 0

        lax.fori_loop(0, n_chunks, body, 0)

    return k(idx_flat, table)


def kernel(indices, table):
    n, s = indices.shape
    b_total = n * s
    assert b_total % (_NW * _CHUNK) == 0
    b_per_w = b_total // _NW
    n_chunks = b_per_w // _CHUNK
    idx_flat = indices.reshape(b_total).astype(jnp.int32)
    out = _gather_flat(idx_flat, table, b_per_w, n_chunks)
    return out.reshape(n, s, table.shape[1])


# speedup vs baseline: 5.1683x; 1.0376x over previous
import functools

import jax
import jax.numpy as jnp
from jax import lax
from jax.experimental import pallas as pl
from jax.experimental.pallas import tpu as pltpu
from jax.experimental.pallas import tpu_sc as plsc

_NC = 2
_NS = 16
_NW = _NC * _NS

_CHUNK = 512
_NBUF = 2


@functools.partial(jax.jit, static_argnums=(2, 3))
def _gather_flat(idx_flat, table, b_per_w, n_chunks):
    d = table.shape[1]
    mesh = plsc.VectorSubcoreMesh(core_axis_name="c", subcore_axis_name="s")
    n_outer = n_chunks // _NBUF

    @functools.partial(
        pl.kernel,
        mesh=mesh,
        out_type=jax.ShapeDtypeStruct((idx_flat.shape[0], d), jnp.float32),
        scratch_types=[
            pltpu.VMEM((_NBUF, _CHUNK), jnp.int32),
            pltpu.VMEM((_NBUF, _CHUNK, d), jnp.float32),
        ]
        + [pltpu.SemaphoreType.DMA] * (3 * _NBUF),
        compiler_params=pltpu.CompilerParams(use_tc_tiling_on_sc=False),
    )
    def k(idx_hbm, table_hbm, out_hbm, idx_v, rows_v, *sems):
        idx_sem = sems[0:_NBUF]
        gat_sem = sems[_NBUF:2 * _NBUF]
        st_sem = sems[2 * _NBUF:3 * _NBUF]
        wid = lax.axis_index("s") * _NC + lax.axis_index("c")
        base = wid * b_per_w

        for b in range(_NBUF):
            pltpu.async_copy(
                idx_hbm.at[pl.ds(base + b * _CHUNK, _CHUNK)],
                idx_v.at[b], idx_sem[b])

        def outer(g, _):
            for b in range(_NBUF):
                c = g * _NBUF + b
                off = base + c * _CHUNK
                pltpu.make_async_copy(
                    idx_hbm.at[pl.ds(off, _CHUNK)], idx_v.at[b],
                    idx_sem[b]).wait()
                @pl.when(g > 0)
                def _():
                    pltpu.make_async_copy(
                        rows_v.at[b], out_hbm.at[pl.ds(off, _CHUNK)],
                        st_sem[b]).wait()
                pltpu.async_copy(table_hbm.at[idx_v.at[b]], rows_v.at[b],
                                 gat_sem[b])
                pltpu.make_async_copy(table_hbm.at[idx_v.at[b]],
                                      rows_v.at[b], gat_sem[b]).wait()
                pltpu.async_copy(rows_v.at[b],
                                 out_hbm.at[pl.ds(off, _CHUNK)], st_sem[b])
                @pl.when(c + _NBUF < n_chunks)
                def _():
                    pltpu.async_copy(
                        idx_hbm.at[pl.ds(off + _NBUF * _CHUNK, _CHUNK)],
                        idx_v.at[b], idx_sem[b])
            return 0

        lax.fori_loop(0, n_outer, outer, 0)

        for b in range(_NBUF):
            pltpu.make_async_copy(
                rows_v.at[b], out_hbm.at[pl.ds(base, _CHUNK)],
                st_sem[b]).wait()

    return k(idx_flat, table)


def kernel(indices, table):
    n, s = indices.shape
    b_total = n * s
    assert b_total % (_NW * _CHUNK * _NBUF) == 0
    b_per_w = b_total // _NW
    n_chunks = b_per_w // _CHUNK
    idx_flat = indices.reshape(b_total).astype(jnp.int32)
    out = _gather_flat(idx_flat, table, b_per_w, n_chunks)
    return out.reshape(n, s, table.shape[1])


# speedup vs baseline: 6.8523x; 1.3258x over previous
import functools

import jax
import jax.numpy as jnp
from jax import lax
from jax.experimental import pallas as pl
from jax.experimental.pallas import tpu as pltpu
from jax.experimental.pallas import tpu_sc as plsc

_NC = 2
_NS = 16
_NW = _NC * _NS

_CHUNK = 400
_NBUF = 2


@functools.partial(jax.jit, static_argnums=(2, 3))
def _gather_flat(idx_flat, table128, b_per_w, n_chunks):
    d = 64
    mesh = plsc.VectorSubcoreMesh(core_axis_name="c", subcore_axis_name="s")
    n_outer = n_chunks // _NBUF

    @functools.partial(
        pl.kernel,
        mesh=mesh,
        out_type=jax.ShapeDtypeStruct((idx_flat.shape[0], 128), jnp.float32),
        scratch_types=[
            pltpu.VMEM((_NBUF * _CHUNK,), jnp.int32),
            pltpu.VMEM((_NBUF, _CHUNK, 128), jnp.float32),
        ]
        + [pltpu.SemaphoreType.DMA] * (3 * _NBUF),
    )
    def k(idx_hbm, table_hbm, out_hbm, idx_v, rows_v, *sems):
        idx_sem = sems[0:_NBUF]
        gat_sem = sems[_NBUF:2 * _NBUF]
        st_sem = sems[2 * _NBUF:3 * _NBUF]
        wid = lax.axis_index("s") * _NC + lax.axis_index("c")
        base = wid * b_per_w

        for b in range(_NBUF):
            pltpu.async_copy(
                idx_hbm.at[pl.ds(base + b * _CHUNK, _CHUNK)],
                idx_v.at[pl.ds(b * _CHUNK, _CHUNK)], idx_sem[b])

        def outer(g, _):
            for b in range(_NBUF):
                c = g * _NBUF + b
                off = base + c * _CHUNK
                idx_slice = idx_v.at[pl.ds(b * _CHUNK, _CHUNK)]
                pltpu.make_async_copy(
                    idx_hbm.at[pl.ds(off, _CHUNK)], idx_slice,
                    idx_sem[b]).wait()
                @pl.when(g > 0)
                def _():
                    pltpu.make_async_copy(
                        rows_v.at[b],
                        out_hbm.at[pl.ds(off, _CHUNK)], st_sem[b]).wait()
                pltpu.async_copy(table_hbm.at[idx_slice], rows_v.at[b],
                                 gat_sem[b])
                pltpu.make_async_copy(table_hbm.at[idx_slice],
                                      rows_v.at[b], gat_sem[b]).wait()
                pltpu.async_copy(rows_v.at[b],
                                 out_hbm.at[pl.ds(off, _CHUNK)], st_sem[b])
                @pl.when(c + _NBUF < n_chunks)
                def _():
                    pltpu.async_copy(
                        idx_hbm.at[pl.ds(off + _NBUF * _CHUNK, _CHUNK)],
                        idx_slice, idx_sem[b])
            return 0

        lax.fori_loop(0, n_outer, outer, 0)

        for b in range(_NBUF):
            pltpu.make_async_copy(
                rows_v.at[b],
                out_hbm.at[pl.ds(base, _CHUNK)], st_sem[b]).wait()

    return k(idx_flat, table128)


def kernel(indices, table):
    n, s = indices.shape
    b_total = n * s
    assert b_total % (_NW * _CHUNK * _NBUF) == 0
    b_per_w = b_total // _NW
    n_chunks = b_per_w // _CHUNK
    idx_flat = indices.reshape(b_total).astype(jnp.int32)
    table128 = jnp.pad(table, ((0, 0), (0, 128 - table.shape[1])))
    out = _gather_flat(idx_flat, table128, b_per_w, n_chunks)
    return out[:, : table.shape[1]].reshape(n, s, table.shape[1])


# speedup vs baseline: 9.7985x; 1.4299x over previous
import functools

import jax
import jax.numpy as jnp
from jax import lax
from jax.experimental import pallas as pl
from jax.experimental.pallas import tpu as pltpu
from jax.experimental.pallas import tpu_sc as plsc

_NC = 2
_NS = 16
_NW = _NC * _NS

_CHUNK = 640
_NBUF = 2


@functools.partial(jax.jit, static_argnums=(2, 3))
def _gather_flat(idx_flat, table, b_per_w, n_chunks):
    d = table.shape[1]
    mesh = plsc.VectorSubcoreMesh(core_axis_name="c", subcore_axis_name="s")
    n_outer = n_chunks // _NBUF

    @functools.partial(
        pl.kernel,
        mesh=mesh,
        out_type=jax.ShapeDtypeStruct((idx_flat.shape[0], 128), jnp.float32),
        scratch_types=[
            pltpu.VMEM((_NBUF * _CHUNK,), jnp.int32),
            pltpu.VMEM((_NBUF, _CHUNK, 64), jnp.float32),
        ]
        + [pltpu.SemaphoreType.DMA] * (3 * _NBUF),
        compiler_params=pltpu.CompilerParams(use_tc_tiling_on_sc=False),
    )
    def k(idx_hbm, table_hbm, out_hbm, idx_v, rows_v, *sems):
        idx_sem = sems[0:_NBUF]
        gat_sem = sems[_NBUF:2 * _NBUF]
        st_sem = sems[2 * _NBUF:3 * _NBUF]
        wid = lax.axis_index("s") * _NC + lax.axis_index("c")
        base = wid * b_per_w

        for b in range(_NBUF):
            pltpu.async_copy(
                idx_hbm.at[pl.ds(base + b * _CHUNK, _CHUNK)],
                idx_v.at[pl.ds(b * _CHUNK, _CHUNK)], idx_sem[b])

        def outer(g, _):
            for b in range(_NBUF):
                c = g * _NBUF + b
                off = base + c * _CHUNK
                idx_slice = idx_v.at[pl.ds(b * _CHUNK, _CHUNK)]
                dst = out_hbm.at[pl.ds(off, _CHUNK), pl.ds(0, 64)]
                pltpu.make_async_copy(
                    idx_hbm.at[pl.ds(off, _CHUNK)], idx_slice,
                    idx_sem[b]).wait()
                @pl.when(g > 0)
                def _():
                    pltpu.make_async_copy(rows_v.at[b], dst, st_sem[b]).wait()
                pltpu.async_copy(table_hbm.at[idx_slice], rows_v.at[b],
                                 gat_sem[b])
                pltpu.make_async_copy(table_hbm.at[idx_slice],
                                      rows_v.at[b], gat_sem[b]).wait()
                pltpu.async_copy(rows_v.at[b], dst, st_sem[b])
                @pl.when(c + _NBUF < n_chunks)
                def _():
                    pltpu.async_copy(
                        idx_hbm.at[pl.ds(off + _NBUF * _CHUNK, _CHUNK)],
                        idx_slice, idx_sem[b])
            return 0

        lax.fori_loop(0, n_outer, outer, 0)

        for b in range(_NBUF):
            pltpu.make_async_copy(
                rows_v.at[b],
                out_hbm.at[pl.ds(base, _CHUNK), pl.ds(0, 64)],
                st_sem[b]).wait()

    return k(idx_flat, table)


def kernel(indices, table):
    n, s = indices.shape
    b_total = n * s
    assert b_total % (_NW * _CHUNK * _NBUF) == 0
    b_per_w = b_total // _NW
    n_chunks = b_per_w // _CHUNK
    idx_flat = indices.reshape(b_total).astype(jnp.int32)
    out = _gather_flat(idx_flat, table, b_per_w, n_chunks)
    return out[:, : table.shape[1]].reshape(n, s, table.shape[1])
